# Initial kernel scaffold; baseline (speedup 1.0000x reference)
import jax
import jax.numpy as jnp
from jax.experimental import pallas as pl


def kernel(x, edge_index, W1, b1, W2, b2, W3, b3):
    raise NotImplementedError("write your pallas kernel here")


# speedup vs baseline: 6.4745x; 6.4745x over previous
import functools

import jax
import jax.numpy as jnp
from jax import lax
from jax.experimental import pallas as pl
from jax.experimental.pallas import tpu as pltpu
from jax.experimental.pallas import tpu_sc as plsc

N = 10000
E = 320000
D = 128
NCLS = 16
NPAD = 10112
NSC = 2
NTEC = 16
ROWS_PER_TILE = NPAD // NTEC
NWORKERS = NSC * NTEC
EDGES_PER_TILE = E // NWORKERS
CHUNK = 128
NCHUNK = -(-EDGES_PER_TILE // CHUNK)
EDGES_PAD = NCHUNK * CHUNK

_MESH = plsc.VectorSubcoreMesh(
    core_axis_name="c", subcore_axis_name="s", num_cores=NSC,
    num_subcores=NTEC)
_SC_PARAMS = pltpu.CompilerParams(use_tc_tiling_on_sc=False)


def _make_spmm(width):
  @functools.partial(
      pl.kernel,
      out_type=jax.ShapeDtypeStruct((NSC, NPAD, width), jnp.float32),
      mesh=_MESH,
      compiler_params=_SC_PARAMS,
      scratch_types=[
          pltpu.VMEM((NCHUNK, CHUNK), jnp.int32),
          pltpu.VMEM((NCHUNK, CHUNK), jnp.int32),
          pltpu.VMEM((CHUNK, width), jnp.float32),
          pltpu.VMEM_SHARED((NPAD, width), jnp.float32),
          pltpu.SemaphoreType.DMA,
      ],
  )
  def spmm(table_hbm, srcs_hbm, dsts_hbm, zeros_hbm, out_hbm,
           src_v, dst_v, rows_v, acc_sh, gsem):
    c = lax.axis_index("c")
    s = lax.axis_index("s")
    wid = c * NTEC + s
    r0 = s * ROWS_PER_TILE
    pltpu.sync_copy(zeros_hbm.at[pl.ds(r0, ROWS_PER_TILE)],
                    acc_sh.at[pl.ds(r0, ROWS_PER_TILE)])
    pltpu.sync_copy(srcs_hbm.at[wid], src_v)
    pltpu.sync_copy(dsts_hbm.at[wid], dst_v)
    plsc.subcore_barrier()

    def body(j, carry):
      pltpu.async_copy(table_hbm.at[src_v.at[j]], rows_v, gsem).wait()
      pltpu.sync_copy(rows_v, acc_sh.at[dst_v.at[j]], add=True)
      return carry

    lax.fori_loop(0, NCHUNK, body, 0)
    plsc.subcore_barrier()
    pltpu.sync_copy(acc_sh.at[pl.ds(r0, ROWS_PER_TILE)],
                    out_hbm.at[c, pl.ds(r0, ROWS_PER_TILE)])

  return spmm


_spmm128 = _make_spmm(D)
_spmm16 = _make_spmm(NCLS)


@functools.partial(
    pl.kernel,
    out_type=jax.ShapeDtypeStruct((NSC, NPAD, NCLS), jnp.float32),
    mesh=_MESH,
    compiler_params=_SC_PARAMS,
    scratch_types=[
        pltpu.VMEM((NCHUNK, CHUNK), jnp.int32),
        pltpu.VMEM((CHUNK, NCLS), jnp.float32),
        pltpu.VMEM_SHARED((NPAD, NCLS), jnp.float32),
    ],
)
def _deg_kernel(dsts_hbm, ones_hbm, zeros_hbm, out_hbm, dst_v, ones_v, acc_sh):
  c = lax.axis_index("c")
  s = lax.axis_index("s")
  wid = c * NTEC + s
  r0 = s * ROWS_PER_TILE
  pltpu.sync_copy(zeros_hbm.at[pl.ds(r0, ROWS_PER_TILE)],
                  acc_sh.at[pl.ds(r0, ROWS_PER_TILE)])
  pltpu.sync_copy(dsts_hbm.at[wid], dst_v)
  pltpu.sync_copy(ones_hbm, ones_v)
  plsc.subcore_barrier()

  def body(j, carry):
    pltpu.sync_copy(ones_v, acc_sh.at[dst_v.at[j]], add=True)
    return carry

  lax.fori_loop(0, NCHUNK, body, 0)
  plsc.subcore_barrier()
  pltpu.sync_copy(acc_sh.at[pl.ds(r0, ROWS_PER_TILE)],
                  out_hbm.at[c, pl.ds(r0, ROWS_PER_TILE)])


def _dinv_from(degp):
  deg = degp[0, :, 0:1] + degp[1, :, 0:1]
  return jnp.where(deg > 0, lax.rsqrt(jnp.maximum(deg, 1.0)), 0.0)


def _k0_body(degp_ref, x_ref, out_ref):
  dinv = _dinv_from(degp_ref[...])
  out_ref[0:N, :] = x_ref[...] * dinv[0:N]
  out_ref[N:NPAD, :] = jnp.zeros((NPAD - N, D), jnp.float32)


def _k1_body(degp_ref, p_ref, w_ref, b_ref, out_ref):
  dinv = _dinv_from(degp_ref[...])
  agg = (p_ref[0] + p_ref[1]) * dinv
  h = jnp.maximum(
      jnp.dot(agg, w_ref[...], preferred_element_type=jnp.float32)
      + b_ref[...], 0.0)
  out_ref[0:N, :] = h[0:N] * dinv[0:N]
  out_ref[N:NPAD, :] = jnp.zeros((NPAD - N, D), jnp.float32)


def _k2_body(degp_ref, p_ref, w2_ref, b2_ref, w3_ref, out_ref):
  dinv = _dinv_from(degp_ref[...])
  agg = (p_ref[0] + p_ref[1]) * dinv
  h = jnp.maximum(
      jnp.dot(agg, w2_ref[...], preferred_element_type=jnp.float32)
      + b2_ref[...], 0.0)
  z = jnp.dot(h * dinv, w3_ref[...], preferred_element_type=jnp.float32)
  out_ref[0:N, :] = z[0:N]
  out_ref[N:NPAD, :] = jnp.zeros((NPAD - N, NCLS), jnp.float32)


def _k3_body(degp_ref, p_ref, b_ref, out_ref):
  dinv = _dinv_from(degp_ref[...])
  out_ref[...] = (p_ref[0, 0:N] + p_ref[1, 0:N]) * dinv[0:N] + b_ref[...]


def kernel(x, edge_index, W1, b1, W2, b2, W3, b3):
  src = edge_index[0].astype(jnp.int32).reshape(NWORKERS, EDGES_PER_TILE)
  dst = edge_index[1].astype(jnp.int32).reshape(NWORKERS, EDGES_PER_TILE)
  padi = jnp.full((NWORKERS, EDGES_PAD - EDGES_PER_TILE), N, dtype=jnp.int32)
  src_b = jnp.concatenate([src, padi], axis=1).reshape(NWORKERS, NCHUNK, CHUNK)
  dst_b = jnp.concatenate([dst, padi], axis=1).reshape(NWORKERS, NCHUNK, CHUNK)
  zeros128 = jnp.zeros((NPAD, D), jnp.float32)
  zeros16 = jnp.zeros((NPAD, NCLS), jnp.float32)
  ones16 = jnp.ones((CHUNK, NCLS), jnp.float32)

  degp = _deg_kernel(dst_b, ones16, zeros16)
  hs0 = pl.pallas_call(
      _k0_body, out_shape=jax.ShapeDtypeStruct((NPAD, D), jnp.float32))(
          degp, x)
  p1 = _spmm128(hs0, src_b, dst_b, zeros128)
  hs1 = pl.pallas_call(
      _k1_body, out_shape=jax.ShapeDtypeStruct((NPAD, D), jnp.float32))(
          degp, p1, W1, b1.reshape(1, D))
  p2 = _spmm128(hs1, src_b, dst_b, zeros128)
  z = pl.pallas_call(
      _k2_body, out_shape=jax.ShapeDtypeStruct((NPAD, NCLS), jnp.float32))(
          degp, p2, W2, b2.reshape(1, D), W3)
  p3 = _spmm16(z, src_b, dst_b, zeros16)
  out = pl.pallas_call(
      _k3_body, out_shape=jax.ShapeDtypeStruct((N, NCLS), jnp.float32))(
          degp, p3, b3.reshape(1, NCLS))
  return out


# speedup vs baseline: 7.3976x; 1.1426x over previous
import functools

import jax
import jax.numpy as jnp
from jax import lax
from jax.experimental import pallas as pl
from jax.experimental.pallas import tpu as pltpu
from jax.experimental.pallas import tpu_sc as plsc

N = 10000
E = 320000
D = 128
DH = D // 2
NCLS = 16
NPAD = 10112
NSC = 2
NTEC = 16
ROWS_PER_TILE = NPAD // NTEC
EDGES_PER_TILE = E // NTEC
CHUNK = 128
NBUF = 4
NCHUNK_HALF = 80
NCHUNK_FULL = 2 * NCHUNK_HALF
EDGES_PAD = NCHUNK_FULL * CHUNK

_MESH = plsc.VectorSubcoreMesh(
    core_axis_name="c", subcore_axis_name="s", num_cores=NSC,
    num_subcores=NTEC)
_SC_PARAMS = pltpu.CompilerParams(use_tc_tiling_on_sc=False)


def _make_spmm(width, col_split):
  nch = NCHUNK_FULL if col_split else NCHUNK_HALF
  tbl_shape = (NSC, NPAD, width) if col_split else (NPAD, width)

  @functools.partial(
      pl.kernel,
      out_type=jax.ShapeDtypeStruct((NSC, NPAD, width), jnp.float32),
      mesh=_MESH,
      compiler_params=_SC_PARAMS,
      scratch_types=[
          pltpu.VMEM((nch, CHUNK), jnp.int32),
          pltpu.VMEM((nch, CHUNK), jnp.int32),
          [pltpu.VMEM((CHUNK, width), jnp.float32)] * NBUF,
          pltpu.VMEM_SHARED((NPAD, width), jnp.float32),
          [pltpu.SemaphoreType.DMA] * NBUF,
          [pltpu.SemaphoreType.DMA] * NBUF,
      ],
  )
  def spmm(table_hbm, srcs_hbm, dsts_hbm, zeros_hbm, out_hbm,
           src_v, dst_v, rows, acc_sh, gsem, ssem):
    c = lax.axis_index("c")
    s = lax.axis_index("s")
    r0 = s * ROWS_PER_TILE
    tbl = table_hbm.at[c] if col_split else table_hbm
    ch0 = 0 if col_split else c * NCHUNK_HALF
    pltpu.sync_copy(zeros_hbm.at[pl.ds(r0, ROWS_PER_TILE)],
                    acc_sh.at[pl.ds(r0, ROWS_PER_TILE)])
    pltpu.sync_copy(srcs_hbm.at[s, pl.ds(ch0, nch)], src_v)
    pltpu.sync_copy(dsts_hbm.at[s, pl.ds(ch0, nch)], dst_v)
    plsc.subcore_barrier()

    pltpu.async_copy(tbl.at[src_v.at[0]], rows[0], gsem[0])
    pltpu.async_copy(tbl.at[src_v.at[1]], rows[1], gsem[1])

    def outer(g, carry):
      for b in range(NBUF):
        j = g * NBUF + b
        b2 = (b + 2) % NBUF

        @pl.when(j >= 2)
        def _():
          pltpu.make_async_copy(
              rows[b2], acc_sh.at[dst_v.at[j - 2]], ssem[b2]).wait()

        @pl.when(j + 2 < nch)
        def _():
          pltpu.async_copy(tbl.at[src_v.at[j + 2]], rows[b2], gsem[b2])

        pltpu.make_async_copy(tbl.at[src_v.at[j]], rows[b], gsem[b]).wait()
        pltpu.async_copy(rows[b], acc_sh.at[dst_v.at[j]], ssem[b], add=True)
      return carry

    lax.fori_loop(0, nch // NBUF, outer, 0)
    pltpu.make_async_copy(
        rows[2], acc_sh.at[dst_v.at[nch - 2]], ssem[2]).wait()
    pltpu.make_async_copy(
        rows[3], acc_sh.at[dst_v.at[nch - 1]], ssem[3]).wait()
    plsc.subcore_barrier()
    pltpu.sync_copy(acc_sh.at[pl.ds(r0, ROWS_PER_TILE)],
                    out_hbm.at[c, pl.ds(r0, ROWS_PER_TILE)])

  return spmm


_spmm_col = _make_spmm(DH, col_split=True)
_spmm16 = _make_spmm(NCLS, col_split=False)


@functools.partial(
    pl.kernel,
    out_type=jax.ShapeDtypeStruct((NSC, NPAD, NCLS), jnp.float32),
    mesh=_MESH,
    compiler_params=_SC_PARAMS,
    scratch_types=[
        pltpu.VMEM((NCHUNK_HALF, CHUNK), jnp.int32),
        pltpu.VMEM((CHUNK, NCLS), jnp.float32),
        pltpu.VMEM_SHARED((NPAD, NCLS), jnp.float32),
    ],
)
def _deg_kernel(dsts_hbm, ones_hbm, zeros_hbm, out_hbm, dst_v, ones_v, acc_sh):
  c = lax.axis_index("c")
  s = lax.axis_index("s")
  r0 = s * ROWS_PER_TILE
  pltpu.sync_copy(zeros_hbm.at[pl.ds(r0, ROWS_PER_TILE)],
                  acc_sh.at[pl.ds(r0, ROWS_PER_TILE)])
  pltpu.sync_copy(dsts_hbm.at[s, pl.ds(c * NCHUNK_HALF, NCHUNK_HALF)], dst_v)
  pltpu.sync_copy(ones_hbm, ones_v)
  plsc.subcore_barrier()

  def body(j, carry):
    pltpu.sync_copy(ones_v, acc_sh.at[dst_v.at[j]], add=True)
    return carry

  lax.fori_loop(0, NCHUNK_HALF, body, 0)
  plsc.subcore_barrier()
  pltpu.sync_copy(acc_sh.at[pl.ds(r0, ROWS_PER_TILE)],
                  out_hbm.at[c, pl.ds(r0, ROWS_PER_TILE)])


def _dinv_from(degp):
  deg = degp[0, :, 0:1] + degp[1, :, 0:1]
  return jnp.where(deg > 0, lax.rsqrt(jnp.maximum(deg, 1.0)), 0.0)


def _k0_body(degp_ref, x_ref, out_ref):
  dinv = _dinv_from(degp_ref[...])
  hs = x_ref[...] * dinv[0:N]
  out_ref[0, 0:N, :] = hs[:, 0:DH]
  out_ref[1, 0:N, :] = hs[:, DH:D]
  out_ref[0, N:NPAD, :] = jnp.zeros((NPAD - N, DH), jnp.float32)
  out_ref[1, N:NPAD, :] = jnp.zeros((NPAD - N, DH), jnp.float32)


def _k1_body(degp_ref, p_ref, w_ref, b_ref, out_ref):
  dinv = _dinv_from(degp_ref[...])
  agg = jnp.concatenate([p_ref[0], p_ref[1]], axis=1) * dinv
  h = jnp.maximum(
      jnp.dot(agg, w_ref[...], preferred_element_type=jnp.float32)
      + b_ref[...], 0.0)
  hs = h[0:N] * dinv[0:N]
  out_ref[0, 0:N, :] = hs[:, 0:DH]
  out_ref[1, 0:N, :] = hs[:, DH:D]
  out_ref[0, N:NPAD, :] = jnp.zeros((NPAD - N, DH), jnp.float32)
  out_ref[1, N:NPAD, :] = jnp.zeros((NPAD - N, DH), jnp.float32)


def _k2_body(degp_ref, p_ref, w2_ref, b2_ref, w3_ref, out_ref):
  dinv = _dinv_from(degp_ref[...])
  agg = jnp.concatenate([p_ref[0], p_ref[1]], axis=1) * dinv
  h = jnp.maximum(
      jnp.dot(agg, w2_ref[...], preferred_element_type=jnp.float32)
      + b2_ref[...], 0.0)
  z = jnp.dot(h * dinv, w3_ref[...], preferred_element_type=jnp.float32)
  out_ref[0:N, :] = z[0:N]
  out_ref[N:NPAD, :] = jnp.zeros((NPAD - N, NCLS), jnp.float32)


def _k3_body(degp_ref, p_ref, b_ref, out_ref):
  dinv = _dinv_from(degp_ref[...])
  out_ref[...] = (p_ref[0, 0:N] + p_ref[1, 0:N]) * dinv[0:N] + b_ref[...]


def kernel(x, edge_index, W1, b1, W2, b2, W3, b3):
  src = edge_index[0].astype(jnp.int32).reshape(NTEC, EDGES_PER_TILE)
  dst = edge_index[1].astype(jnp.int32).reshape(NTEC, EDGES_PER_TILE)
  padi = jnp.full((NTEC, EDGES_PAD - EDGES_PER_TILE), N, dtype=jnp.int32)
  src_b = jnp.concatenate([src, padi], axis=1).reshape(
      NTEC, NCHUNK_FULL, CHUNK)
  dst_b = jnp.concatenate([dst, padi], axis=1).reshape(
      NTEC, NCHUNK_FULL, CHUNK)
  zeros64 = jnp.zeros((NPAD, DH), jnp.float32)
  zeros16 = jnp.zeros((NPAD, NCLS), jnp.float32)
  ones16 = jnp.ones((CHUNK, NCLS), jnp.float32)

  degp = _deg_kernel(dst_b, ones16, zeros16)
  hs0 = pl.pallas_call(
      _k0_body, out_shape=jax.ShapeDtypeStruct((NSC, NPAD, DH), jnp.float32))(
          degp, x)
  p1 = _spmm_col(hs0, src_b, dst_b, zeros64)
  hs1 = pl.pallas_call(
      _k1_body, out_shape=jax.ShapeDtypeStruct((NSC, NPAD, DH), jnp.float32))(
          degp, p1, W1, b1.reshape(1, D))
  p2 = _spmm_col(hs1, src_b, dst_b, zeros64)
  z = pl.pallas_call(
      _k2_body, out_shape=jax.ShapeDtypeStruct((NPAD, NCLS), jnp.float32))(
          degp, p2, W2, b2.reshape(1, D), W3)
  p3 = _spmm16(z, src_b, dst_b, zeros16)
  out = pl.pallas_call(
      _k3_body, out_shape=jax.ShapeDtypeStruct((N, NCLS), jnp.float32))(
          degp, p3, b3.reshape(1, NCLS))
  return out


# speedup vs baseline: 11.9650x; 1.6174x over previous
import functools

import jax
import jax.numpy as jnp
from jax import lax
from jax.experimental import pallas as pl
from jax.experimental.pallas import tpu as pltpu
from jax.experimental.pallas import tpu_sc as plsc

N = 10000
E = 320000
D = 128
DH = D // 2
NCLS = 16
NPAD = 10112
NSC = 2
NTEC = 16
ROWS_PER_TILE = NPAD // NTEC
EDGES_PER_TILE = E // NTEC
CHUNK = 128
NBUF = 4
NCH_PH = 80
NCHUNK_HALF = 80
NCHUNK_FULL = 2 * NCHUNK_HALF
EDGES_PAD = NCHUNK_FULL * CHUNK

_MESH = plsc.VectorSubcoreMesh(
    core_axis_name="c", subcore_axis_name="s", num_cores=NSC,
    num_subcores=NTEC)
_SC_PARAMS = pltpu.CompilerParams(use_tc_tiling_on_sc=False)


def _make_spmm(width, col_split):
  nch = NCHUNK_FULL if col_split else NCHUNK_HALF
  nph = nch // NCH_PH

  @functools.partial(
      pl.kernel,
      out_type=jax.ShapeDtypeStruct((NSC, NPAD, width), jnp.float32),
      mesh=_MESH,
      compiler_params=_SC_PARAMS,
      scratch_types=[
          pltpu.VMEM((NCH_PH, CHUNK), jnp.int32),
          [pltpu.VMEM((CHUNK,), jnp.int32)] * NBUF,
          [pltpu.VMEM((CHUNK,), jnp.int32)] * NBUF,
          [pltpu.VMEM((CHUNK, width), jnp.float32)] * NBUF,
          pltpu.VMEM_SHARED((NPAD, width), jnp.float32),
          pltpu.VMEM_SHARED((NPAD, width), jnp.float32),
          [pltpu.SemaphoreType.DMA] * NBUF,
          [pltpu.SemaphoreType.DMA] * NBUF,
      ],
  )
  def spmm(table_hbm, pidx_hbm, zeros_hbm, out_hbm,
           pidx_v, srcb, dstb, rows, tbl_sh, acc_sh, gsem, ssem):
    c = lax.axis_index("c")
    s = lax.axis_index("s")
    r0 = s * ROWS_PER_TILE
    tbl = table_hbm.at[c] if col_split else table_hbm
    ch0 = 0 if col_split else c * NCHUNK_HALF
    pltpu.sync_copy(zeros_hbm.at[pl.ds(r0, ROWS_PER_TILE)],
                    acc_sh.at[pl.ds(r0, ROWS_PER_TILE)])
    pltpu.sync_copy(tbl.at[pl.ds(r0, ROWS_PER_TILE)],
                    tbl_sh.at[pl.ds(r0, ROWS_PER_TILE)])
    plsc.subcore_barrier()

    def unpack(t, b):
      for k in range(CHUNK // 16):
        v = pidx_v[t, pl.ds(k * 16, 16)]
        srcb[b][pl.ds(k * 16, 16)] = lax.bitwise_and(v, 0xFFFF)
        dstb[b][pl.ds(k * 16, 16)] = lax.shift_right_logical(v, 16)

    for ph in range(nph):
      pltpu.sync_copy(pidx_hbm.at[s, pl.ds(ch0 + ph * NCH_PH, NCH_PH)],
                      pidx_v)
      for k in range(2):
        unpack(k, k)
        pltpu.async_copy(tbl_sh.at[srcb[k]], rows[k], gsem[k])

      def outer(g, carry):
        for b in range(NBUF):
          t = g * NBUF + b
          b2 = (b + 2) % NBUF

          @pl.when(t >= 2)
          def _():
            pltpu.make_async_copy(
                rows[b2], acc_sh.at[dstb[b2]], ssem[b2]).wait()

          @pl.when(t + 2 < NCH_PH)
          def _():
            unpack(t + 2, b2)
            pltpu.async_copy(tbl_sh.at[srcb[b2]], rows[b2], gsem[b2])

          pltpu.make_async_copy(tbl_sh.at[srcb[b]], rows[b], gsem[b]).wait()
          pltpu.async_copy(rows[b], acc_sh.at[dstb[b]], ssem[b], add=True)
        return carry

      lax.fori_loop(0, NCH_PH // NBUF, outer, 0)
      pltpu.make_async_copy(
          rows[2], acc_sh.at[dstb[2]], ssem[2]).wait()
      pltpu.make_async_copy(
          rows[3], acc_sh.at[dstb[3]], ssem[3]).wait()
    plsc.subcore_barrier()
    pltpu.sync_copy(acc_sh.at[pl.ds(r0, ROWS_PER_TILE)],
                    out_hbm.at[c, pl.ds(r0, ROWS_PER_TILE)])

  return spmm


_spmm_col = _make_spmm(DH, col_split=True)
_spmm16 = _make_spmm(NCLS, col_split=False)


@functools.partial(
    pl.kernel,
    out_type=jax.ShapeDtypeStruct((NSC, NPAD, NCLS), jnp.float32),
    mesh=_MESH,
    compiler_params=_SC_PARAMS,
    scratch_types=[
        pltpu.VMEM((NCHUNK_HALF, CHUNK), jnp.int32),
        pltpu.VMEM((CHUNK, NCLS), jnp.float32),
        pltpu.VMEM_SHARED((NPAD, NCLS), jnp.float32),
    ],
)
def _deg_kernel(dsts_hbm, ones_hbm, zeros_hbm, out_hbm, dst_v, ones_v, acc_sh):
  c = lax.axis_index("c")
  s = lax.axis_index("s")
  r0 = s * ROWS_PER_TILE
  pltpu.sync_copy(zeros_hbm.at[pl.ds(r0, ROWS_PER_TILE)],
                  acc_sh.at[pl.ds(r0, ROWS_PER_TILE)])
  pltpu.sync_copy(dsts_hbm.at[s, pl.ds(c * NCHUNK_HALF, NCHUNK_HALF)], dst_v)
  pltpu.sync_copy(ones_hbm, ones_v)
  plsc.subcore_barrier()

  def body(j, carry):
    pltpu.sync_copy(ones_v, acc_sh.at[dst_v.at[j]], add=True)
    return carry

  lax.fori_loop(0, NCHUNK_HALF, body, 0)
  plsc.subcore_barrier()
  pltpu.sync_copy(acc_sh.at[pl.ds(r0, ROWS_PER_TILE)],
                  out_hbm.at[c, pl.ds(r0, ROWS_PER_TILE)])


def _dinv_from(degp):
  deg = degp[0, :, 0:1] + degp[1, :, 0:1]
  return jnp.where(deg > 0, lax.rsqrt(jnp.maximum(deg, 1.0)), 0.0)


def _k0_body(degp_ref, x_ref, out_ref):
  dinv = _dinv_from(degp_ref[...])
  hs = x_ref[...] * dinv[0:N]
  out_ref[0, 0:N, :] = hs[:, 0:DH]
  out_ref[1, 0:N, :] = hs[:, DH:D]
  out_ref[0, N:NPAD, :] = jnp.zeros((NPAD - N, DH), jnp.float32)
  out_ref[1, N:NPAD, :] = jnp.zeros((NPAD - N, DH), jnp.float32)


def _k1_body(degp_ref, p_ref, w_ref, b_ref, out_ref):
  dinv = _dinv_from(degp_ref[...])
  agg = jnp.concatenate([p_ref[0], p_ref[1]], axis=1) * dinv
  h = jnp.maximum(
      jnp.dot(agg, w_ref[...], preferred_element_type=jnp.float32)
      + b_ref[...], 0.0)
  hs = h[0:N] * dinv[0:N]
  out_ref[0, 0:N, :] = hs[:, 0:DH]
  out_ref[1, 0:N, :] = hs[:, DH:D]
  out_ref[0, N:NPAD, :] = jnp.zeros((NPAD - N, DH), jnp.float32)
  out_ref[1, N:NPAD, :] = jnp.zeros((NPAD - N, DH), jnp.float32)


def _k2_body(degp_ref, p_ref, w2_ref, b2_ref, w3_ref, out_ref):
  dinv = _dinv_from(degp_ref[...])
  agg = jnp.concatenate([p_ref[0], p_ref[1]], axis=1) * dinv
  h = jnp.maximum(
      jnp.dot(agg, w2_ref[...], preferred_element_type=jnp.float32)
      + b2_ref[...], 0.0)
  z = jnp.dot(h * dinv, w3_ref[...], preferred_element_type=jnp.float32)
  out_ref[0:N, :] = z[0:N]
  out_ref[N:NPAD, :] = jnp.zeros((NPAD - N, NCLS), jnp.float32)


def _k3_body(degp_ref, p_ref, b_ref, out_ref):
  dinv = _dinv_from(degp_ref[...])
  out_ref[...] = (p_ref[0, 0:N] + p_ref[1, 0:N]) * dinv[0:N] + b_ref[...]


def kernel(x, edge_index, W1, b1, W2, b2, W3, b3):
  src = edge_index[0].astype(jnp.int32).reshape(NTEC, EDGES_PER_TILE)
  dst = edge_index[1].astype(jnp.int32).reshape(NTEC, EDGES_PER_TILE)
  padi = jnp.full((NTEC, EDGES_PAD - EDGES_PER_TILE), N, dtype=jnp.int32)
  src_b = jnp.concatenate([src, padi], axis=1).reshape(
      NTEC, NCHUNK_FULL, CHUNK)
  dst_b = jnp.concatenate([dst, padi], axis=1).reshape(
      NTEC, NCHUNK_FULL, CHUNK)
  pidx_b = jnp.bitwise_or(src_b, jnp.left_shift(dst_b, 16))
  zeros64 = jnp.zeros((NPAD, DH), jnp.float32)
  zeros16 = jnp.zeros((NPAD, NCLS), jnp.float32)
  ones16 = jnp.ones((CHUNK, NCLS), jnp.float32)

  degp = _deg_kernel(dst_b, ones16, zeros16)
  hs0 = pl.pallas_call(
      _k0_body, out_shape=jax.ShapeDtypeStruct((NSC, NPAD, DH), jnp.float32))(
          degp, x)
  p1 = _spmm_col(hs0, pidx_b, zeros64)
  hs1 = pl.pallas_call(
      _k1_body, out_shape=jax.ShapeDtypeStruct((NSC, NPAD, DH), jnp.float32))(
          degp, p1, W1, b1.reshape(1, D))
  p2 = _spmm_col(hs1, pidx_b, zeros64)
  z = pl.pallas_call(
      _k2_body, out_shape=jax.ShapeDtypeStruct((NPAD, NCLS), jnp.float32))(
          degp, p2, W2, b2.reshape(1, D), W3)
  p3 = _spmm16(z, pidx_b, zeros16)
  out = pl.pallas_call(
      _k3_body, out_shape=jax.ShapeDtypeStruct((N, NCLS), jnp.float32))(
          degp, p3, b3.reshape(1, NCLS))
  return out


# speedup vs baseline: 16.4605x; 1.3757x over previous
import functools

import jax
import jax.numpy as jnp
from jax import lax
from jax.experimental import pallas as pl
from jax.experimental.pallas import tpu as pltpu
from jax.experimental.pallas import tpu_sc as plsc

N = 10000
E = 320000
D = 128
DH = D // 2
NCLS = 16
NPAD = 10112
NSC = 2
NTEC = 16
ROWS_PER_TILE = NPAD // NTEC
CHUNK = 128
NBUF = 4
NCH_ALL = 2560
E_PAD = NCH_ALL * CHUNK
NCH_COL = NCH_ALL // NTEC
NCH_EDGE = NCH_ALL // (NSC * NTEC)

_MESH = plsc.VectorSubcoreMesh(
    core_axis_name="c", subcore_axis_name="s", num_cores=NSC,
    num_subcores=NTEC)
_SC_PARAMS = pltpu.CompilerParams(use_tc_tiling_on_sc=False)


def _make_spmm(width, col_split, dtype):
  nch = NCH_COL if col_split else NCH_EDGE

  @functools.partial(
      pl.kernel,
      out_type=jax.ShapeDtypeStruct((NSC, NPAD, width), dtype),
      mesh=_MESH,
      compiler_params=_SC_PARAMS,
      scratch_types=[
          pltpu.VMEM((nch, CHUNK), jnp.int32),
          pltpu.VMEM((nch, CHUNK), jnp.int32),
          [pltpu.VMEM((CHUNK, width), dtype)] * NBUF,
          pltpu.VMEM_SHARED((NPAD, width), dtype),
          pltpu.VMEM_SHARED((NPAD, width), dtype),
          [pltpu.SemaphoreType.DMA] * NBUF,
          [pltpu.SemaphoreType.DMA] * NBUF,
      ],
  )
  def spmm(table_hbm, edges_hbm, zeros_hbm, out_hbm,
           src_v, dst_v, rows, tbl_sh, acc_sh, gsem, ssem):
    c = lax.axis_index("c")
    s = lax.axis_index("s")
    r0 = s * ROWS_PER_TILE
    tbl = table_hbm.at[c] if col_split else table_hbm
    c0 = s * NCH_COL if col_split else (c * NTEC + s) * NCH_EDGE
    pltpu.sync_copy(zeros_hbm.at[pl.ds(r0, ROWS_PER_TILE)],
                    acc_sh.at[pl.ds(r0, ROWS_PER_TILE)])
    pltpu.sync_copy(tbl.at[pl.ds(r0, ROWS_PER_TILE)],
                    tbl_sh.at[pl.ds(r0, ROWS_PER_TILE)])
    pltpu.sync_copy(edges_hbm.at[0, pl.ds(c0, nch)], src_v)
    pltpu.sync_copy(edges_hbm.at[1, pl.ds(c0, nch)], dst_v)
    plsc.subcore_barrier()

    pltpu.async_copy(tbl_sh.at[src_v.at[0]], rows[0], gsem[0])
    pltpu.async_copy(tbl_sh.at[src_v.at[1]], rows[1], gsem[1])

    def outer(g, carry):
      for b in range(NBUF):
        t = g * NBUF + b
        b2 = (b + 2) % NBUF

        @pl.when(t >= 2)
        def _():
          pltpu.make_async_copy(
              rows[b2], acc_sh.at[dst_v.at[t - 2]], ssem[b2]).wait()

        @pl.when(t + 2 < nch)
        def _():
          pltpu.async_copy(tbl_sh.at[src_v.at[t + 2]], rows[b2], gsem[b2])

        pltpu.make_async_copy(tbl_sh.at[src_v.at[t]], rows[b], gsem[b]).wait()
        pltpu.async_copy(rows[b], acc_sh.at[dst_v.at[t]], ssem[b], add=True)
      return carry

    lax.fori_loop(0, nch // NBUF, outer, 0)
    pltpu.make_async_copy(
        rows[2], acc_sh.at[dst_v.at[nch - 2]], ssem[2]).wait()
    pltpu.make_async_copy(
        rows[3], acc_sh.at[dst_v.at[nch - 1]], ssem[3]).wait()
    plsc.subcore_barrier()
    pltpu.sync_copy(acc_sh.at[pl.ds(r0, ROWS_PER_TILE)],
                    out_hbm.at[c, pl.ds(r0, ROWS_PER_TILE)])

  return spmm


_spmm_col = _make_spmm(DH, col_split=True, dtype=jnp.bfloat16)
_spmm16 = _make_spmm(NCLS, col_split=False, dtype=jnp.float32)


@functools.partial(
    pl.kernel,
    out_type=jax.ShapeDtypeStruct((NSC, NPAD, NCLS), jnp.float32),
    mesh=_MESH,
    compiler_params=_SC_PARAMS,
    scratch_types=[
        pltpu.VMEM((NCH_EDGE, CHUNK), jnp.int32),
        pltpu.VMEM((CHUNK, NCLS), jnp.float32),
        pltpu.VMEM_SHARED((NPAD, NCLS), jnp.float32),
        [pltpu.SemaphoreType.DMA] * NBUF,
    ],
)
def _deg_kernel(edges_hbm, ones_hbm, zeros_hbm, out_hbm,
                dst_v, ones_v, acc_sh, ssem):
  c = lax.axis_index("c")
  s = lax.axis_index("s")
  r0 = s * ROWS_PER_TILE
  c0 = (c * NTEC + s) * NCH_EDGE
  pltpu.sync_copy(zeros_hbm.at[pl.ds(r0, ROWS_PER_TILE)],
                  acc_sh.at[pl.ds(r0, ROWS_PER_TILE)])
  pltpu.sync_copy(edges_hbm.at[1, pl.ds(c0, NCH_EDGE)], dst_v)
  pltpu.sync_copy(ones_hbm, ones_v)
  plsc.subcore_barrier()

  def outer(g, carry):
    for b in range(NBUF):
      t = g * NBUF + b

      @pl.when(t >= NBUF)
      def _():
        pltpu.make_async_copy(ones_v, acc_sh.at[dst_v.at[t - NBUF]],
                              ssem[b]).wait()

      pltpu.async_copy(ones_v, acc_sh.at[dst_v.at[t]], ssem[b], add=True)
    return carry

  lax.fori_loop(0, NCH_EDGE // NBUF, outer, 0)
  for b in range(NBUF):
    pltpu.make_async_copy(ones_v, acc_sh.at[dst_v.at[NCH_EDGE - NBUF + b]],
                          ssem[b]).wait()
  plsc.subcore_barrier()
  pltpu.sync_copy(acc_sh.at[pl.ds(r0, ROWS_PER_TILE)],
                  out_hbm.at[c, pl.ds(r0, ROWS_PER_TILE)])


def _kprep_body(ei_ref, out_ref):
  out_ref[:, 0:E] = ei_ref[...].astype(jnp.int32)
  out_ref[:, E:E_PAD] = jnp.full((2, E_PAD - E), N, jnp.int32)


GBLK = NPAD // 8
OBLK = N // 10


def _dinv_blk(degp):
  deg = degp[0, :, 0:1] + degp[1, :, 0:1]
  return jnp.where(deg > 0, lax.rsqrt(jnp.maximum(deg, 1.0)), 0.0)


def _rowmask(g):
  rows = g * GBLK + lax.broadcasted_iota(jnp.int32, (GBLK, 1), 0)
  return rows < N


def _k0_body(degp_ref, x_ref, out_ref):
  g = pl.program_id(0)
  dinv = jnp.where(_rowmask(g), _dinv_blk(degp_ref[...]), 0.0)
  hs = (x_ref[...] * dinv).astype(jnp.bfloat16)
  out_ref[0] = hs[:, 0:DH]
  out_ref[1] = hs[:, DH:D]


def _k1_body(degp_ref, p_ref, w_ref, b_ref, out_ref):
  g = pl.program_id(0)
  dinv = _dinv_blk(degp_ref[...])
  agg = jnp.concatenate(
      [p_ref[0], p_ref[1]], axis=1).astype(jnp.float32) * dinv
  h = jnp.maximum(
      jnp.dot(agg, w_ref[...], preferred_element_type=jnp.float32)
      + b_ref[...], 0.0)
  hs = (h * jnp.where(_rowmask(g), dinv, 0.0)).astype(jnp.bfloat16)
  out_ref[0] = hs[:, 0:DH]
  out_ref[1] = hs[:, DH:D]


def _k2_body(degp_ref, p_ref, w2_ref, b2_ref, w3_ref, out_ref):
  g = pl.program_id(0)
  dinv = _dinv_blk(degp_ref[...])
  agg = jnp.concatenate(
      [p_ref[0], p_ref[1]], axis=1).astype(jnp.float32) * dinv
  h = jnp.maximum(
      jnp.dot(agg, w2_ref[...], preferred_element_type=jnp.float32)
      + b2_ref[...], 0.0)
  out_ref[...] = jnp.dot(h * jnp.where(_rowmask(g), dinv, 0.0), w3_ref[...],
                         preferred_element_type=jnp.float32)


def _k3_body(degp_ref, p_ref, b_ref, out_ref):
  deg = degp_ref[0, :, 0:1] + degp_ref[1, :, 0:1]
  dinv = jnp.where(deg > 0, lax.rsqrt(jnp.maximum(deg, 1.0)), 0.0)
  out_ref[...] = (p_ref[0] + p_ref[1]) * dinv + b_ref[...]


def _blk3(width):
  return pl.BlockSpec((NSC, GBLK, width), lambda g: (0, g, 0))


def _full(shape):
  return pl.BlockSpec(shape, lambda g: (0,) * len(shape))


def kernel(x, edge_index, W1, b1, W2, b2, W3, b3):
  edges_b = pl.pallas_call(
      _kprep_body,
      out_shape=jax.ShapeDtypeStruct((2, E_PAD), jnp.int32))(
          edge_index).reshape(2, NCH_ALL, CHUNK)
  x_pad = jnp.pad(x, ((0, NPAD - N), (0, 0)))
  zeros64 = jnp.zeros((NPAD, DH), jnp.bfloat16)
  zeros16 = jnp.zeros((NPAD, NCLS), jnp.float32)
  ones16 = jnp.ones((CHUNK, NCLS), jnp.float32)
  grid = NPAD // GBLK

  degp = _deg_kernel(edges_b, ones16, zeros16)
  hs0 = pl.pallas_call(
      _k0_body,
      grid=(grid,),
      in_specs=[_blk3(NCLS), pl.BlockSpec((GBLK, D), lambda g: (g, 0))],
      out_specs=_blk3(DH),
      out_shape=jax.ShapeDtypeStruct((NSC, NPAD, DH), jnp.bfloat16))(
          degp, x_pad)
  p1 = _spmm_col(hs0, edges_b, zeros64)
  hs1 = pl.pallas_call(
      _k1_body,
      grid=(grid,),
      in_specs=[_blk3(NCLS), _blk3(DH), _full((D, D)), _full((D,))],
      out_specs=_blk3(DH),
      out_shape=jax.ShapeDtypeStruct((NSC, NPAD, DH), jnp.bfloat16))(
          degp, p1, W1, b1)
  p2 = _spmm_col(hs1, edges_b, zeros64)
  z = pl.pallas_call(
      _k2_body,
      grid=(grid,),
      in_specs=[_blk3(NCLS), _blk3(DH), _full((D, D)), _full((D,)),
                _full((D, NCLS))],
      out_specs=pl.BlockSpec((GBLK, NCLS), lambda g: (g, 0)),
      out_shape=jax.ShapeDtypeStruct((NPAD, NCLS), jnp.float32))(
          degp, p2, W2, b2, W3)
  p3 = _spmm16(z, edges_b, zeros16)
  out = pl.pallas_call(
      _k3_body,
      grid=(N // OBLK,),
      in_specs=[pl.BlockSpec((NSC, OBLK, NCLS), lambda g: (0, g, 0)),
                pl.BlockSpec((NSC, OBLK, NCLS), lambda g: (0, g, 0)),
                _full((NCLS,))],
      out_specs=pl.BlockSpec((OBLK, NCLS), lambda g: (g, 0)),
      out_shape=jax.ShapeDtypeStruct((N, NCLS), jnp.float32))(
          degp, p3, b3)
  return out
